# baseline (speedup 1.0000x reference)
import functools

import jax
import jax.numpy as jnp
import numpy as np
from jax import lax
from jax.experimental import pallas as pl
from jax.experimental.pallas import tpu as pltpu
from jax.experimental.pallas import tpu_sc as plsc

D = 128
NC = 2
NS = 16
NW = NC * NS
L = 16

CH = 80
NCH_E = 125
NTC = 4
EBLK = 4000


def _round_up(x, m):
    return (x + m - 1) // m * m


_PERM = np.concatenate(
    [np.stack([np.arange(16), 16 + np.arange(16)], 1).reshape(-1) + 32 * g
     for g in range(D // 32)])


def _dense_body(e_ref, q_ref, rel_ref, went_ref, wq_ref, wedge_ref,
                tn_ref, ta_ref, tc_ref, rq_ref, qt_ref):
    we = wedge_ref[...]
    w1 = we[0 * D:1 * D]
    w2 = we[1 * D:2 * D]
    w3 = we[2 * D:3 * D]
    w4 = we[3 * D:4 * D]
    nt = jnp.dot(e_ref[...], went_ref[...], preferred_element_type=jnp.float32)
    tn_ref[...] = nt
    ta_ref[...] = jnp.dot(
        nt, w1, preferred_element_type=jnp.float32).astype(jnp.bfloat16)
    tc_ref[...] = jnp.dot(
        nt, w3, preferred_element_type=jnp.float32).astype(jnp.bfloat16)

    @pl.when(pl.program_id(0) == 0)
    def _():
        qt = jnp.dot(q_ref[...], wq_ref[...], preferred_element_type=jnp.float32)
        qt_ref[...] = qt
        qp = jnp.dot(qt, w4, preferred_element_type=jnp.float32)
        rp = jnp.dot(rel_ref[...], w2, preferred_element_type=jnp.float32)
        b = qp.shape[0]
        n_rel = rp.shape[0]
        rq = (qp[:, None, :] + rp[None, :, :]).reshape(b * n_rel, D)
        rq_ref[...] = rq.astype(jnp.bfloat16)


def _dense(entity_table, question_emb, relation_table, W_ent, W_q, W_edge):
    n_ent = entity_table.shape[0]
    b = question_emb.shape[0]
    n_rel = relation_table.shape[0]
    grid = (n_ent // EBLK,)
    blk = lambda i: (i, 0)
    zero = lambda i: (0, 0)
    return pl.pallas_call(
        _dense_body,
        grid=grid,
        in_specs=[
            pl.BlockSpec((EBLK, D), blk),
            pl.BlockSpec((b, D), zero),
            pl.BlockSpec((n_rel, D), zero),
            pl.BlockSpec((D, D), zero),
            pl.BlockSpec((D, D), zero),
            pl.BlockSpec((4 * D, D), zero),
        ],
        out_specs=[
            pl.BlockSpec((EBLK, D), blk),
            pl.BlockSpec((EBLK, D), blk),
            pl.BlockSpec((EBLK, D), blk),
            pl.BlockSpec((b * n_rel, D), zero),
            pl.BlockSpec((b, D), zero),
        ],
        out_shape=[
            jax.ShapeDtypeStruct((n_ent, D), jnp.float32),
            jax.ShapeDtypeStruct((n_ent, D), jnp.bfloat16),
            jax.ShapeDtypeStruct((n_ent, D), jnp.bfloat16),
            jax.ShapeDtypeStruct((b * n_rel, D), jnp.bfloat16),
            jax.ShapeDtypeStruct((b, D), jnp.float32),
        ],
    )(entity_table, question_emb, relation_table, W_ent, W_q, W_edge)


def _make_edge_body(n_rel):
    def _edge_body(ta_hbm, rq_hbm, tc_hbm, tn_hbm, h_hbm, t_hbm, r_hbm,
                   b_hbm, ids4_hbm, emb_hbm, gid_hbm,
                   out_hbm, hg_hbm, tg_hbm, nt_hbm,
                   idxb, rqib, heb, teb, ab, rqb, tcb, cb, ntb, hgb, tgb,
                   emb_v, gid_v, semg, semo, semi):
        w = lax.axis_index("s") * NC + lax.axis_index("c")
        pltpu.sync_copy(emb_hbm, emb_v)
        pltpu.sync_copy(gid_hbm, gid_v)
        srcs = (h_hbm, t_hbm, r_hbm, b_hbm)

        def fire_idx(g, slot):
            for k in range(4):
                pltpu.async_copy(srcs[k].at[pl.ds(g * CH, CH)],
                                 idxb.at[slot, k], semi.at[slot])

        def drain_idx(slot):
            for k in range(4):
                pltpu.make_async_copy(srcs[k].at[pl.ds(0, CH)],
                                      idxb.at[slot, k], semi.at[slot]).wait()

        def stage(g, t):
            for j in range(CH // L):
                s = pl.ds(j * L, L)
                rqib[t, s] = idxb[t, 3, s] * n_rel + idxb[t, 2, s]
                heb[t, s] = plsc.load_gather(emb_v, [idxb[t, 0, s]])
                teb[t, s] = plsc.load_gather(emb_v, [idxb[t, 1, s]])
            pltpu.async_copy(ta_hbm.at[heb.at[t]], ab.at[t], semg.at[t])
            pltpu.async_copy(rq_hbm.at[rqib.at[t]], rqb.at[t], semg.at[t])
            pltpu.async_copy(tc_hbm.at[teb.at[t]], tcb.at[t], semg.at[t])

        def drain_gathers(t):
            pltpu.make_async_copy(ta_hbm.at[heb.at[t]], ab.at[t],
                                  semg.at[t]).wait()
            pltpu.make_async_copy(rq_hbm.at[rqib.at[t]], rqb.at[t],
                                  semg.at[t]).wait()
            pltpu.make_async_copy(tc_hbm.at[teb.at[t]], tcb.at[t],
                                  semg.at[t]).wait()

        def drain_out(t):
            pltpu.make_async_copy(cb.at[t], out_hbm.at[pl.ds(0, CH)],
                                  semo.at[t]).wait()
            pltpu.make_async_copy(hgb.at[t], hg_hbm.at[pl.ds(0, CH)],
                                  semo.at[t]).wait()
            pltpu.make_async_copy(tgb.at[t], tg_hbm.at[pl.ds(0, CH)],
                                  semo.at[t]).wait()

        def compute(t):
            @plsc.parallel_loop(0, CH, 1, unroll=4)
            def edge_e(e):
                for g in range(D // 32):
                    s2 = pl.ds(g * L, L)
                    wa = plsc.bitcast(ab[t, e, s2], jnp.bfloat16)
                    wr = plsc.bitcast(rqb[t, e, s2], jnp.bfloat16)
                    wc = plsc.bitcast(tcb[t, e, s2], jnp.bfloat16)
                    ua, va = plsc.unpack(wa,
                                         format=plsc.PackFormat.INTERLEAVED)
                    ur, vr = plsc.unpack(wr,
                                         format=plsc.PackFormat.INTERLEAVED)
                    uc, vc = plsc.unpack(wc,
                                         format=plsc.PackFormat.INTERLEAVED)
                    cb[t, e, pl.ds(g * 32, L)] = ua + ur + uc
                    cb[t, e, pl.ds(g * 32 + L, L)] = va + vr + vc

            @plsc.parallel_loop(0, CH // L, 1, unroll=2)
            def g16(j):
                s = pl.ds(j * L, L)
                hgb[t, s] = plsc.load_gather(gid_v, [idxb[t, 0, s]])
                tgb[t, s] = plsc.load_gather(gid_v, [idxb[t, 1, s]])

        def issue_out(g, t):
            base = g * CH
            pltpu.async_copy(cb.at[t], out_hbm.at[pl.ds(base, CH)], semo.at[t])
            pltpu.async_copy(hgb.at[t], hg_hbm.at[pl.ds(base, CH)], semo.at[t])
            pltpu.async_copy(tgb.at[t], tg_hbm.at[pl.ds(base, CH)], semo.at[t])

        g0 = w * NCH_E
        for k in range(4):
            pltpu.sync_copy(srcs[k].at[pl.ds(g0 * CH, CH)], idxb.at[0, k])
        fire_idx(g0 + 1, 1)
        stage(g0, 0)

        def body(i, carry):
            for t in range(3):
                j = 3 * i + t
                g = g0 + j
                nxt = (t + 1) % 3
                prv = (t + 2) % 3
                drain_gathers(t)
                if t == 2:
                    drain_out(nxt)
                else:
                    @pl.when(i >= 1)
                    def _():
                        drain_out(nxt)
                drain_idx(nxt)
                stage(g + 1, nxt)
                fire_idx(g + 2, prv)
                compute(t)
                issue_out(g, t)
            return carry

        lax.fori_loop(0, (NCH_E - 2) // 3, body, 0)

        drain_gathers(0)
        drain_out(1)
        drain_idx(1)
        stage(g0 + NCH_E - 1, 1)
        compute(0)
        issue_out(g0 + NCH_E - 2, 0)

        drain_gathers(1)
        drain_out(2)
        compute(1)
        issue_out(g0 + NCH_E - 1, 1)

        drain_out(0)
        drain_out(1)

        pltpu.sync_copy(ids4_hbm.at[w], idxb.at[0])
        dsts = (cb.at[0], cb.at[1], cb.at[2], ntb)
        for k in range(NTC):
            pltpu.async_copy(tn_hbm.at[idxb.at[0, k]], dsts[k], semg.at[0])
        for k in range(NTC):
            pltpu.make_async_copy(tn_hbm.at[idxb.at[0, k]], dsts[k],
                                  semg.at[0]).wait()
            pltpu.sync_copy(dsts[k], nt_hbm.at[pl.ds(w * NTC * CH + k * CH, CH)])

    return _edge_body


def _edge_stage(TA, RQ, TC_, TN, heads, tails, rels, batch, ids4, emb_ids,
                gids, n_rel, ne_pad, n_node_pad):
    mesh = plsc.VectorSubcoreMesh(core_axis_name="c", subcore_axis_name="s")
    n_nodes = gids.shape[0]
    f = functools.partial(
        pl.kernel,
        out_type=(
            jax.ShapeDtypeStruct((ne_pad, D), jnp.float32),
            jax.ShapeDtypeStruct((ne_pad,), jnp.int32),
            jax.ShapeDtypeStruct((ne_pad,), jnp.int32),
            jax.ShapeDtypeStruct((n_node_pad, D), jnp.float32),
        ),
        mesh=mesh,
        scratch_types=[
            pltpu.VMEM((3, 4, CH), jnp.int32),
            pltpu.VMEM((3, CH), jnp.int32),
            pltpu.VMEM((3, CH), jnp.int32),
            pltpu.VMEM((3, CH), jnp.int32),
            pltpu.VMEM((3, CH, D // 2), jnp.int32),
            pltpu.VMEM((3, CH, D // 2), jnp.int32),
            pltpu.VMEM((3, CH, D // 2), jnp.int32),
            pltpu.VMEM((3, CH, D), jnp.float32),
            pltpu.VMEM((CH, D), jnp.float32),
            pltpu.VMEM((3, CH), jnp.int32),
            pltpu.VMEM((3, CH), jnp.int32),
            pltpu.VMEM((n_nodes,), jnp.int32),
            pltpu.VMEM((n_nodes,), jnp.int32),
            pltpu.SemaphoreType.DMA((3,)),
            pltpu.SemaphoreType.DMA((3,)),
            pltpu.SemaphoreType.DMA((3,)),
        ],
        compiler_params=pltpu.CompilerParams(needs_layout_passes=False,
                                             use_tc_tiling_on_sc=False),
    )(_make_edge_body(n_rel))
    return f(TA, RQ, TC_, TN, heads, tails, rels, batch, ids4, emb_ids, gids)


def kernel(question_emb, entity_table, relation_table, W_ent, W_q, W_edge,
           node_embedding_ids, node_global_ids, edge_index, edge_relations,
           edge_batch):
    n_nodes = node_embedding_ids.shape[0]
    n_edges = edge_relations.shape[0]
    n_rel = relation_table.shape[0]
    n_node_pad = _round_up(n_nodes, NW * NTC * CH)
    ne_pad = NW * NCH_E * CH
    assert ne_pad == n_edges

    ids4 = jnp.pad(node_embedding_ids,
                   (0, n_node_pad - n_nodes)).reshape(NW, NTC, CH)

    TN, TA, TC_, RQ, qt = _dense(
        entity_table, question_emb, relation_table, W_ent, W_q,
        W_edge[:, _PERM])

    as_i32 = lambda x: jax.lax.bitcast_convert_type(
        x.reshape(x.shape[0], D // 2, 2), jnp.int32)

    edge_tok, hg, tg, nt_pad = _edge_stage(
        as_i32(TA), as_i32(RQ), as_i32(TC_), TN, edge_index[0],
        edge_index[1], edge_relations, edge_batch, ids4,
        node_embedding_ids, node_global_ids, n_rel, ne_pad, n_node_pad)

    return (edge_tok[:n_edges], nt_pad[:n_nodes], qt,
            hg[:n_edges], tg[:n_edges])


# speedup vs baseline: 3.2007x; 3.2007x over previous
import functools

import jax
import jax.numpy as jnp
from jax import lax
from jax.experimental import pallas as pl
from jax.experimental.pallas import tpu as pltpu
from jax.experimental.pallas import tpu_sc as plsc

D = 128
NC = 2
NS = 16
NW = NC * NS
L = 16

CH = 80
NCH_E = 125
NTC = 4
EBLK = 4000


def _round_up(x, m):
    return (x + m - 1) // m * m


def _dense_body(e_ref, q_ref, rel_ref, went_ref, wq_ref, wedge_ref,
                tn_ref, ta_ref, tc_ref, rq_ref, qt_ref):
    we = wedge_ref[...]
    w1 = we[0 * D:1 * D]
    w2 = we[1 * D:2 * D]
    w3 = we[2 * D:3 * D]
    w4 = we[3 * D:4 * D]
    nt = jnp.dot(e_ref[...], went_ref[...], preferred_element_type=jnp.float32)
    tn_ref[...] = nt
    ta_ref[...] = jnp.dot(nt, w1, preferred_element_type=jnp.float32)
    tc_ref[...] = jnp.dot(nt, w3, preferred_element_type=jnp.float32)

    @pl.when(pl.program_id(0) == 0)
    def _():
        qt = jnp.dot(q_ref[...], wq_ref[...], preferred_element_type=jnp.float32)
        qt_ref[...] = qt
        qp = jnp.dot(qt, w4, preferred_element_type=jnp.float32)
        rp = jnp.dot(rel_ref[...], w2, preferred_element_type=jnp.float32)
        b = qp.shape[0]
        n_rel = rp.shape[0]
        rq_ref[...] = (qp[:, None, :] + rp[None, :, :]).reshape(b * n_rel, D)


def _dense(entity_table, question_emb, relation_table, W_ent, W_q, W_edge):
    n_ent = entity_table.shape[0]
    b = question_emb.shape[0]
    n_rel = relation_table.shape[0]
    grid = (n_ent // EBLK,)
    blk = lambda i: (i, 0)
    zero = lambda i: (0, 0)
    return pl.pallas_call(
        _dense_body,
        grid=grid,
        in_specs=[
            pl.BlockSpec((EBLK, D), blk),
            pl.BlockSpec((b, D), zero),
            pl.BlockSpec((n_rel, D), zero),
            pl.BlockSpec((D, D), zero),
            pl.BlockSpec((D, D), zero),
            pl.BlockSpec((4 * D, D), zero),
        ],
        out_specs=[
            pl.BlockSpec((EBLK, D), blk),
            pl.BlockSpec((EBLK, D), blk),
            pl.BlockSpec((EBLK, D), blk),
            pl.BlockSpec((b * n_rel, D), zero),
            pl.BlockSpec((b, D), zero),
        ],
        out_shape=[
            jax.ShapeDtypeStruct((n_ent, D), jnp.float32),
            jax.ShapeDtypeStruct((n_ent, D), jnp.float32),
            jax.ShapeDtypeStruct((n_ent, D), jnp.float32),
            jax.ShapeDtypeStruct((b * n_rel, D), jnp.float32),
            jax.ShapeDtypeStruct((b, D), jnp.float32),
        ],
    )(entity_table, question_emb, relation_table, W_ent, W_q, W_edge)


def _make_edge_body(n_rel):
    def _edge_body(ta_hbm, rq_hbm, tc_hbm, tn_hbm, h_hbm, t_hbm, r_hbm,
                   b_hbm, ids4_hbm, emb_hbm, gid_hbm,
                   out_hbm, hg_hbm, tg_hbm, nt_hbm,
                   idxb, rqib, heb, teb, ab, rqb, cb, hgb, tgb,
                   emb_v, gid_v, semg, semo, semi):
        w = lax.axis_index("s") * NC + lax.axis_index("c")
        pltpu.sync_copy(emb_hbm, emb_v)
        pltpu.sync_copy(gid_hbm, gid_v)
        srcs = (h_hbm, t_hbm, r_hbm, b_hbm)

        def fire_idx(g, slot):
            for k in range(4):
                pltpu.async_copy(srcs[k].at[pl.ds(g * CH, CH)],
                                 idxb.at[slot, k], semi.at[slot])

        def drain_idx(slot):
            for k in range(4):
                pltpu.make_async_copy(srcs[k].at[pl.ds(0, CH)],
                                      idxb.at[slot, k], semi.at[slot]).wait()

        def stage(g, t):
            for j in range(CH // L):
                s = pl.ds(j * L, L)
                rqib[t, s] = idxb[t, 3, s] * n_rel + idxb[t, 2, s]
                heb[t, s] = plsc.load_gather(emb_v, [idxb[t, 0, s]])
                teb[t, s] = plsc.load_gather(emb_v, [idxb[t, 1, s]])
            pltpu.async_copy(ta_hbm.at[heb.at[t]], ab.at[t], semg.at[t])
            pltpu.async_copy(rq_hbm.at[rqib.at[t]], rqb.at[t], semg.at[t])
            pltpu.async_copy(tc_hbm.at[teb.at[t]], cb.at[t], semg.at[t])

        def drain_gathers(t):
            pltpu.make_async_copy(ta_hbm.at[heb.at[t]], ab.at[t],
                                  semg.at[t]).wait()
            pltpu.make_async_copy(rq_hbm.at[rqib.at[t]], rqb.at[t],
                                  semg.at[t]).wait()
            pltpu.make_async_copy(tc_hbm.at[teb.at[t]], cb.at[t],
                                  semg.at[t]).wait()

        def drain_out(t):
            pltpu.make_async_copy(cb.at[t], out_hbm.at[pl.ds(0, CH)],
                                  semo.at[t]).wait()
            pltpu.make_async_copy(hgb.at[t], hg_hbm.at[pl.ds(0, CH)],
                                  semo.at[t]).wait()
            pltpu.make_async_copy(tgb.at[t], tg_hbm.at[pl.ds(0, CH)],
                                  semo.at[t]).wait()

        def compute(t):
            @plsc.parallel_loop(0, CH, 1, unroll=8)
            def edge_e(e):
                for col in range(D // L):
                    s = pl.ds(col * L, L)
                    cb[t, e, s] = ab[t, e, s] + rqb[t, e, s] + cb[t, e, s]

            @plsc.parallel_loop(0, CH // L, 1, unroll=2)
            def g16(j):
                s = pl.ds(j * L, L)
                hgb[t, s] = plsc.load_gather(gid_v, [idxb[t, 0, s]])
                tgb[t, s] = plsc.load_gather(gid_v, [idxb[t, 1, s]])

        def issue_out(g, t):
            base = g * CH
            pltpu.async_copy(cb.at[t], out_hbm.at[pl.ds(base, CH)], semo.at[t])
            pltpu.async_copy(hgb.at[t], hg_hbm.at[pl.ds(base, CH)], semo.at[t])
            pltpu.async_copy(tgb.at[t], tg_hbm.at[pl.ds(base, CH)], semo.at[t])

        g0 = w * NCH_E
        for k in range(4):
            pltpu.sync_copy(srcs[k].at[pl.ds(g0 * CH, CH)], idxb.at[0, k])
        fire_idx(g0 + 1, 1)
        stage(g0, 0)

        def body(i, carry):
            for t in range(3):
                j = 3 * i + t
                g = g0 + j
                nxt = (t + 1) % 3
                prv = (t + 2) % 3
                drain_gathers(t)
                if t == 2:
                    drain_out(nxt)
                else:
                    @pl.when(i >= 1)
                    def _():
                        drain_out(nxt)
                drain_idx(nxt)
                stage(g + 1, nxt)
                fire_idx(g + 2, prv)
                compute(t)
                issue_out(g, t)
            return carry

        lax.fori_loop(0, (NCH_E - 2) // 3, body, 0)

        drain_gathers(0)
        drain_out(1)
        drain_idx(1)
        stage(g0 + NCH_E - 1, 1)
        compute(0)
        issue_out(g0 + NCH_E - 2, 0)

        drain_gathers(1)
        drain_out(2)
        compute(1)
        issue_out(g0 + NCH_E - 1, 1)

        drain_out(0)
        drain_out(1)

        pltpu.sync_copy(ids4_hbm.at[w], idxb.at[0])
        dsts = (ab.at[0], ab.at[1], ab.at[2], rqb.at[0])
        for k in range(NTC):
            pltpu.async_copy(tn_hbm.at[idxb.at[0, k]], dsts[k], semg.at[0])
        for k in range(NTC):
            pltpu.make_async_copy(tn_hbm.at[idxb.at[0, k]], dsts[k],
                                  semg.at[0]).wait()
            pltpu.sync_copy(dsts[k], nt_hbm.at[pl.ds(w * NTC * CH + k * CH, CH)])

    return _edge_body


def _edge_stage(TA, RQ, TC_, TN, heads, tails, rels, batch, ids4, emb_ids,
                gids, n_rel, ne_pad, n_node_pad):
    mesh = plsc.VectorSubcoreMesh(core_axis_name="c", subcore_axis_name="s")
    n_nodes = gids.shape[0]
    f = functools.partial(
        pl.kernel,
        out_type=(
            jax.ShapeDtypeStruct((ne_pad, D), jnp.float32),
            jax.ShapeDtypeStruct((ne_pad,), jnp.int32),
            jax.ShapeDtypeStruct((ne_pad,), jnp.int32),
            jax.ShapeDtypeStruct((n_node_pad, D), jnp.float32),
        ),
        mesh=mesh,
        scratch_types=[
            pltpu.VMEM((3, 4, CH), jnp.int32),
            pltpu.VMEM((3, CH), jnp.int32),
            pltpu.VMEM((3, CH), jnp.int32),
            pltpu.VMEM((3, CH), jnp.int32),
            pltpu.VMEM((3, CH, D), jnp.float32),
            pltpu.VMEM((3, CH, D), jnp.float32),
            pltpu.VMEM((3, CH, D), jnp.float32),
            pltpu.VMEM((3, CH), jnp.int32),
            pltpu.VMEM((3, CH), jnp.int32),
            pltpu.VMEM((n_nodes,), jnp.int32),
            pltpu.VMEM((n_nodes,), jnp.int32),
            pltpu.SemaphoreType.DMA((3,)),
            pltpu.SemaphoreType.DMA((3,)),
            pltpu.SemaphoreType.DMA((3,)),
        ],
        compiler_params=pltpu.CompilerParams(needs_layout_passes=False),
    )(_make_edge_body(n_rel))
    return f(TA, RQ, TC_, TN, heads, tails, rels, batch, ids4, emb_ids, gids)


def kernel(question_emb, entity_table, relation_table, W_ent, W_q, W_edge,
           node_embedding_ids, node_global_ids, edge_index, edge_relations,
           edge_batch):
    n_nodes = node_embedding_ids.shape[0]
    n_edges = edge_relations.shape[0]
    n_rel = relation_table.shape[0]
    n_node_pad = _round_up(n_nodes, NW * NTC * CH)
    ne_pad = NW * NCH_E * CH
    assert ne_pad == n_edges

    ids4 = jnp.pad(node_embedding_ids,
                   (0, n_node_pad - n_nodes)).reshape(NW, NTC, CH)

    TN, TA, TC_, RQ, qt = _dense(
        entity_table, question_emb, relation_table, W_ent, W_q, W_edge)

    edge_tok, hg, tg, nt_pad = _edge_stage(
        TA, RQ, TC_, TN, edge_index[0], edge_index[1], edge_relations,
        edge_batch, ids4, node_embedding_ids, node_global_ids,
        n_rel, ne_pad, n_node_pad)

    return (edge_tok[:n_edges], nt_pad[:n_nodes], qt,
            hg[:n_edges], tg[:n_edges])


# speedup vs baseline: 3.2037x; 1.0009x over previous
import functools

import jax
import jax.numpy as jnp
from jax import lax
from jax.experimental import pallas as pl
from jax.experimental.pallas import tpu as pltpu
from jax.experimental.pallas import tpu_sc as plsc

D = 128
NC = 2
NS = 16
NW = NC * NS
L = 16

CH = 80
NCH_E = 125
NTC = 4
EBLK = 4000


def _round_up(x, m):
    return (x + m - 1) // m * m


def _dense_body(e_ref, q_ref, rel_ref, went_ref, wq_ref, wedge_ref,
                tn_ref, ta_ref, tc_ref, rq_ref, qt_ref):
    we = wedge_ref[...]
    w1 = we[0 * D:1 * D]
    w2 = we[1 * D:2 * D]
    w3 = we[2 * D:3 * D]
    w4 = we[3 * D:4 * D]
    nt = jnp.dot(e_ref[...], went_ref[...], preferred_element_type=jnp.float32)
    tn_ref[...] = nt
    ta_ref[...] = jnp.dot(nt, w1, preferred_element_type=jnp.float32)
    tc_ref[...] = jnp.dot(nt, w3, preferred_element_type=jnp.float32)

    @pl.when(pl.program_id(0) == 0)
    def _():
        qt = jnp.dot(q_ref[...], wq_ref[...], preferred_element_type=jnp.float32)
        qt_ref[...] = qt
        qp = jnp.dot(qt, w4, preferred_element_type=jnp.float32)
        rp = jnp.dot(rel_ref[...], w2, preferred_element_type=jnp.float32)
        b = qp.shape[0]
        n_rel = rp.shape[0]
        rq_ref[...] = (qp[:, None, :] + rp[None, :, :]).reshape(b * n_rel, D)


def _dense(entity_table, question_emb, relation_table, W_ent, W_q, W_edge):
    n_ent = entity_table.shape[0]
    b = question_emb.shape[0]
    n_rel = relation_table.shape[0]
    grid = (n_ent // EBLK,)
    blk = lambda i: (i, 0)
    zero = lambda i: (0, 0)
    return pl.pallas_call(
        _dense_body,
        grid=grid,
        in_specs=[
            pl.BlockSpec((EBLK, D), blk),
            pl.BlockSpec((b, D), zero),
            pl.BlockSpec((n_rel, D), zero),
            pl.BlockSpec((D, D), zero),
            pl.BlockSpec((D, D), zero),
            pl.BlockSpec((4 * D, D), zero),
        ],
        out_specs=[
            pl.BlockSpec((EBLK, D), blk),
            pl.BlockSpec((EBLK, D), blk),
            pl.BlockSpec((EBLK, D), blk),
            pl.BlockSpec((b * n_rel, D), zero),
            pl.BlockSpec((b, D), zero),
        ],
        out_shape=[
            jax.ShapeDtypeStruct((n_ent, D), jnp.float32),
            jax.ShapeDtypeStruct((n_ent, D), jnp.float32),
            jax.ShapeDtypeStruct((n_ent, D), jnp.float32),
            jax.ShapeDtypeStruct((b * n_rel, D), jnp.float32),
            jax.ShapeDtypeStruct((b, D), jnp.float32),
        ],
    )(entity_table, question_emb, relation_table, W_ent, W_q, W_edge)


def _make_edge_body(n_rel):
    def _edge_body(ta_hbm, rq_hbm, tc_hbm, tn_hbm, h_hbm, t_hbm, r_hbm,
                   b_hbm, ids4_hbm, emb_hbm, gid_hbm,
                   out_hbm, hg_hbm, tg_hbm, nt_hbm,
                   idxb, rqib, heb, teb, ab, rqb, cb, hgb, tgb,
                   emb_v, gid_v, semg, semo, semi):
        w = lax.axis_index("s") * NC + lax.axis_index("c")
        pltpu.sync_copy(emb_hbm, emb_v)
        pltpu.sync_copy(gid_hbm, gid_v)
        srcs = (h_hbm, t_hbm, r_hbm, b_hbm)

        def fire_idx(g, slot):
            for k in range(4):
                pltpu.async_copy(srcs[k].at[pl.ds(g * CH, CH)],
                                 idxb.at[slot, k], semi.at[slot])

        def drain_idx(slot):
            for k in range(4):
                pltpu.make_async_copy(srcs[k].at[pl.ds(0, CH)],
                                      idxb.at[slot, k], semi.at[slot]).wait()

        def stage(g, t):
            for j in range(CH // L):
                s = pl.ds(j * L, L)
                rqib[t, s] = idxb[t, 3, s] * n_rel + idxb[t, 2, s]
                heb[t, s] = plsc.load_gather(emb_v, [idxb[t, 0, s]])
                teb[t, s] = plsc.load_gather(emb_v, [idxb[t, 1, s]])
            pltpu.async_copy(ta_hbm.at[heb.at[t]], ab.at[t], semg.at[t])
            pltpu.async_copy(rq_hbm.at[rqib.at[t]], rqb.at[t], semg.at[t])
            pltpu.async_copy(tc_hbm.at[teb.at[t]], cb.at[t], semg.at[t])

        def drain_gathers(t):
            pltpu.make_async_copy(ta_hbm.at[heb.at[t]], ab.at[t],
                                  semg.at[t]).wait()
            pltpu.make_async_copy(rq_hbm.at[rqib.at[t]], rqb.at[t],
                                  semg.at[t]).wait()
            pltpu.make_async_copy(tc_hbm.at[teb.at[t]], cb.at[t],
                                  semg.at[t]).wait()

        def drain_out(t):
            pltpu.make_async_copy(cb.at[t], out_hbm.at[pl.ds(0, CH)],
                                  semo.at[t]).wait()
            pltpu.make_async_copy(hgb.at[t], hg_hbm.at[pl.ds(0, CH)],
                                  semo.at[t]).wait()
            pltpu.make_async_copy(tgb.at[t], tg_hbm.at[pl.ds(0, CH)],
                                  semo.at[t]).wait()

        def compute(t):
            @plsc.parallel_loop(0, CH, 1, unroll=4)
            def edge_e(e):
                for col in range(D // L):
                    s = pl.ds(col * L, L)
                    cb[t, e, s] = ab[t, e, s] + rqb[t, e, s] + cb[t, e, s]

            @plsc.parallel_loop(0, CH // L, 1, unroll=2)
            def g16(j):
                s = pl.ds(j * L, L)
                hgb[t, s] = plsc.load_gather(gid_v, [idxb[t, 0, s]])
                tgb[t, s] = plsc.load_gather(gid_v, [idxb[t, 1, s]])

        def issue_out(g, t):
            base = g * CH
            pltpu.async_copy(cb.at[t], out_hbm.at[pl.ds(base, CH)], semo.at[t])
            pltpu.async_copy(hgb.at[t], hg_hbm.at[pl.ds(base, CH)], semo.at[t])
            pltpu.async_copy(tgb.at[t], tg_hbm.at[pl.ds(base, CH)], semo.at[t])

        g0 = w * NCH_E
        for k in range(4):
            pltpu.sync_copy(srcs[k].at[pl.ds(g0 * CH, CH)], idxb.at[0, k])
        fire_idx(g0 + 1, 1)
        stage(g0, 0)

        def body(i, carry):
            for t in range(3):
                j = 3 * i + t
                g = g0 + j
                nxt = (t + 1) % 3
                prv = (t + 2) % 3
                drain_gathers(t)
                if t == 2:
                    drain_out(nxt)
                else:
                    @pl.when(i >= 1)
                    def _():
                        drain_out(nxt)
                drain_idx(nxt)
                stage(g + 1, nxt)
                fire_idx(g + 2, prv)
                compute(t)
                issue_out(g, t)
            return carry

        lax.fori_loop(0, (NCH_E - 2) // 3, body, 0)

        drain_gathers(0)
        drain_out(1)
        drain_idx(1)
        stage(g0 + NCH_E - 1, 1)
        compute(0)
        issue_out(g0 + NCH_E - 2, 0)

        drain_gathers(1)
        drain_out(2)
        compute(1)
        issue_out(g0 + NCH_E - 1, 1)

        drain_out(0)
        drain_out(1)

        pltpu.sync_copy(ids4_hbm.at[w], idxb.at[0])
        dsts = (ab.at[0], ab.at[1], ab.at[2], rqb.at[0])
        for k in range(NTC):
            pltpu.async_copy(tn_hbm.at[idxb.at[0, k]], dsts[k], semg.at[0])
        for k in range(NTC):
            pltpu.make_async_copy(tn_hbm.at[idxb.at[0, k]], dsts[k],
                                  semg.at[0]).wait()
            pltpu.sync_copy(dsts[k], nt_hbm.at[pl.ds(w * NTC * CH + k * CH, CH)])

    return _edge_body


def _edge_stage(TA, RQ, TC_, TN, heads, tails, rels, batch, ids4, emb_ids,
                gids, n_rel, ne_pad, n_node_pad):
    mesh = plsc.VectorSubcoreMesh(core_axis_name="c", subcore_axis_name="s")
    n_nodes = gids.shape[0]
    f = functools.partial(
        pl.kernel,
        out_type=(
            jax.ShapeDtypeStruct((ne_pad, D), jnp.float32),
            jax.ShapeDtypeStruct((ne_pad,), jnp.int32),
            jax.ShapeDtypeStruct((ne_pad,), jnp.int32),
            jax.ShapeDtypeStruct((n_node_pad, D), jnp.float32),
        ),
        mesh=mesh,
        scratch_types=[
            pltpu.VMEM((3, 4, CH), jnp.int32),
            pltpu.VMEM((3, CH), jnp.int32),
            pltpu.VMEM((3, CH), jnp.int32),
            pltpu.VMEM((3, CH), jnp.int32),
            pltpu.VMEM((3, CH, D), jnp.float32),
            pltpu.VMEM((3, CH, D), jnp.float32),
            pltpu.VMEM((3, CH, D), jnp.float32),
            pltpu.VMEM((3, CH), jnp.int32),
            pltpu.VMEM((3, CH), jnp.int32),
            pltpu.VMEM((n_nodes,), jnp.int32),
            pltpu.VMEM((n_nodes,), jnp.int32),
            pltpu.SemaphoreType.DMA((3,)),
            pltpu.SemaphoreType.DMA((3,)),
            pltpu.SemaphoreType.DMA((3,)),
        ],
        compiler_params=pltpu.CompilerParams(needs_layout_passes=False),
    )(_make_edge_body(n_rel))
    return f(TA, RQ, TC_, TN, heads, tails, rels, batch, ids4, emb_ids, gids)


def kernel(question_emb, entity_table, relation_table, W_ent, W_q, W_edge,
           node_embedding_ids, node_global_ids, edge_index, edge_relations,
           edge_batch):
    n_nodes = node_embedding_ids.shape[0]
    n_edges = edge_relations.shape[0]
    n_rel = relation_table.shape[0]
    n_node_pad = _round_up(n_nodes, NW * NTC * CH)
    ne_pad = NW * NCH_E * CH
    assert ne_pad == n_edges

    ids4 = jnp.pad(node_embedding_ids,
                   (0, n_node_pad - n_nodes)).reshape(NW, NTC, CH)

    TN, TA, TC_, RQ, qt = _dense(
        entity_table, question_emb, relation_table, W_ent, W_q, W_edge)

    edge_tok, hg, tg, nt_pad = _edge_stage(
        TA, RQ, TC_, TN, edge_index[0], edge_index[1], edge_relations,
        edge_batch, ids4, node_embedding_ids, node_global_ids,
        n_rel, ne_pad, n_node_pad)

    return (edge_tok[:n_edges], nt_pad[:n_nodes], qt,
            hg[:n_edges], tg[:n_edges])
